# Initial kernel scaffold; baseline (speedup 1.0000x reference)
import jax
import jax.numpy as jnp
from jax.experimental import pallas as pl


def kernel(x, edge_index, W_map, b_map, Wl1, bl1, Wr1, Wl2, bl2, Wr2, Wl3, bl3, Wr3):
    raise NotImplementedError("write your pallas kernel here")


# speedup vs baseline: 4.0384x; 4.0384x over previous
import functools

import jax
import jax.numpy as jnp
from jax import lax
from jax.experimental import pallas as pl
from jax.experimental.pallas import tpu as pltpu
from jax.experimental.pallas import tpu_sc as plsc

N = 10000
E = 320000
F = 128
H = 128
H2 = 256
C = 40

NC = 2
NS = 16
LANES = 16
NW = NC * NS

K = 128
E_PAD = 327680
EPT = E_PAD // NW
NCHUNK = EPT // K
NPAD = 10240
RPT = NPAD // NS
ZR = 64

D1 = 144
D2 = 128
D3 = 48

BN = 400
GRID = N // BN

_PREC = lax.Precision.HIGHEST


def _dot(a, b):
    return jnp.dot(a, b, precision=_PREC, preferred_element_type=jnp.float32)


def _make_sc_pass(D):
    mesh = plsc.VectorSubcoreMesh(core_axis_name="c", subcore_axis_name="s")

    @functools.partial(
        pl.kernel,
        mesh=mesh,
        compiler_params=pltpu.CompilerParams(use_tc_tiling_on_sc=False),
        out_type=jax.ShapeDtypeStruct((NC, NPAD, D), jnp.float32),
        scratch_types=[
            pltpu.VMEM((2, K), jnp.int32),
            pltpu.VMEM((K, D), jnp.float32),
            pltpu.VMEM((ZR, D), jnp.float32),
            pltpu.VMEM_SHARED((NPAD, D), jnp.float32),
            pltpu.SemaphoreType.DMA,
        ],
    )
    def sc_pass(table_hbm, edge_hbm, out_hbm, ev, rows_v, zero_v, acc_sh, sem):
        c = lax.axis_index("c")
        s = lax.axis_index("s")
        wid = c * NS + s

        @pl.loop(0, ZR)
        def _zrow(r):
            @pl.loop(0, D, step=LANES)
            def _zcol(col):
                zero_v[r, pl.ds(col, LANES)] = jnp.zeros((LANES,), jnp.float32)

        zbase = s * RPT

        @pl.loop(0, RPT, step=ZR)
        def _zero(r0):
            pltpu.sync_copy(zero_v, acc_sh.at[pl.ds(zbase + r0, ZR)])

        plsc.subcore_barrier()

        ebase = wid * EPT

        @pl.loop(0, NCHUNK)
        def _chunk(ci):
            off = ebase + ci * K
            pltpu.sync_copy(edge_hbm.at[:, pl.ds(off, K)], ev)
            pltpu.async_copy(table_hbm.at[ev.at[0]], rows_v, sem).wait()
            pltpu.sync_copy(rows_v, acc_sh.at[ev.at[1]], add=True)

        plsc.subcore_barrier()

        pltpu.sync_copy(acc_sh.at[pl.ds(zbase, RPT)],
                        out_hbm.at[c].at[pl.ds(zbase, RPT)])

    return sc_pass


_sc_pass_d1 = _make_sc_pass(D1)
_sc_pass_d2 = _make_sc_pass(D2)
_sc_pass_d3 = _make_sc_pass(D3)


def _row_spec(d):
    return pl.BlockSpec((BN, d), lambda i: (i, 0))


def _acc_spec(d):
    return pl.BlockSpec((NC, BN, d), lambda i: (0, i, 0))


def _full_spec(shape):
    return pl.BlockSpec(shape, lambda i: tuple(0 for _ in shape))


def _pre_body(x_ref, wmap_ref, bmap_ref, wl1_ref, wr1_ref,
              h0_ref, t1_ref, u1_ref):
    h0 = _dot(x_ref[...], wmap_ref[...]) + bmap_ref[...]
    h0_ref[...] = h0
    p1 = _dot(h0, wl1_ref[...])
    t1_ref[...] = jnp.concatenate(
        [p1, jnp.ones((BN, D1 - H), jnp.float32)], axis=1)
    u1_ref[...] = _dot(h0, wr1_ref[...])


def _l1_body(acc_ref, u1_ref, bl1_ref, h1_ref, rdeg_ref):
    ssum = acc_ref[0, :, :H] + acc_ref[1, :, :H]
    deg = acc_ref[0, :, H:H + 8] + acc_ref[1, :, H:H + 8]
    rdeg = 1.0 / jnp.maximum(deg[:, :1], 1.0)
    h1 = ssum * rdeg + bl1_ref[...] + u1_ref[...]
    h1_ref[...] = jnp.maximum(h1, 0.0)
    rdeg_ref[...] = jnp.broadcast_to(rdeg, (BN, 8))


def _mm_body(a_ref, w_ref, o_ref):
    o_ref[...] = _dot(a_ref[...], w_ref[...])


def _l2_body(acc_ref, rdeg_ref, v2_ref, wl2_ref, bl2_ref, wl3_ref,
             h2_ref, t3_ref):
    agg = (acc_ref[0] + acc_ref[1]) * rdeg_ref[:, :1]
    h2 = _dot(agg, wl2_ref[...]) + bl2_ref[...] + v2_ref[...]
    h2 = jnp.maximum(h2, 0.0)
    h2_ref[...] = h2
    p3 = _dot(h2, wl3_ref[...])
    t3_ref[...] = jnp.concatenate(
        [p3, jnp.zeros((BN, D3 - C), jnp.float32)], axis=1)


def _l3_body(acc_ref, rdeg_ref, v3_ref, bl3_ref, out_ref):
    aggs = acc_ref[0] + acc_ref[1]
    z = aggs[:, :C] * rdeg_ref[:, :1] + bl3_ref[...] + v3_ref[...]
    m = jnp.max(z, axis=1, keepdims=True)
    lse = jnp.log(jnp.sum(jnp.exp(z - m), axis=1, keepdims=True)) + m
    out_ref[...] = z - lse


def kernel(x, edge_index, W_map, b_map, Wl1, bl1, Wr1, Wl2, bl2, Wr2,
           Wl3, bl3, Wr3):
    edges = edge_index.astype(jnp.int32)
    pad = jnp.tile(jnp.array([[0], [NPAD - 1]], jnp.int32), (1, E_PAD - E))
    edges = jnp.concatenate([edges, pad], axis=1)

    h0, t1, u1 = pl.pallas_call(
        _pre_body,
        grid=(GRID,),
        in_specs=[_row_spec(F), _full_spec((F, H)), _full_spec((1, H)),
                  _full_spec((H, H)), _full_spec((H, H))],
        out_specs=[_row_spec(H), _row_spec(D1), _row_spec(H)],
        out_shape=[jax.ShapeDtypeStruct((N, H), jnp.float32),
                   jax.ShapeDtypeStruct((N, D1), jnp.float32),
                   jax.ShapeDtypeStruct((N, H), jnp.float32)],
    )(x, W_map, b_map.reshape(1, H), Wl1, Wr1)

    acc1 = _sc_pass_d1(t1, edges)

    h1, rdeg = pl.pallas_call(
        _l1_body,
        grid=(GRID,),
        in_specs=[_acc_spec(D1), _row_spec(H), _full_spec((1, H))],
        out_specs=[_row_spec(H), _row_spec(8)],
        out_shape=[jax.ShapeDtypeStruct((N, H), jnp.float32),
                   jax.ShapeDtypeStruct((N, 8), jnp.float32)],
    )(acc1, u1, bl1.reshape(1, H))

    acc2 = _sc_pass_d2(h1, edges)

    v2 = pl.pallas_call(
        _mm_body,
        grid=(GRID,),
        in_specs=[_row_spec(H), _full_spec((H, H2))],
        out_specs=_row_spec(H2),
        out_shape=jax.ShapeDtypeStruct((N, H2), jnp.float32),
    )(h1, Wr2)

    h2, t3 = pl.pallas_call(
        _l2_body,
        grid=(GRID,),
        in_specs=[_acc_spec(D2), _row_spec(8), _row_spec(H2),
                  _full_spec((H, H2)), _full_spec((1, H2)),
                  _full_spec((H2, C))],
        out_specs=[_row_spec(H2), _row_spec(D3)],
        out_shape=[jax.ShapeDtypeStruct((N, H2), jnp.float32),
                   jax.ShapeDtypeStruct((N, D3), jnp.float32)],
    )(acc2, rdeg, v2, Wl2, bl2.reshape(1, H2), Wl3)

    acc3 = _sc_pass_d3(t3, edges)

    v3 = pl.pallas_call(
        _mm_body,
        grid=(GRID,),
        in_specs=[_row_spec(H2), _full_spec((H2, C))],
        out_specs=_row_spec(C),
        out_shape=jax.ShapeDtypeStruct((N, C), jnp.float32),
    )(h2, Wr3)

    out = pl.pallas_call(
        _l3_body,
        grid=(GRID,),
        in_specs=[_acc_spec(D3), _row_spec(8), _row_spec(C),
                  _full_spec((1, C))],
        out_specs=_row_spec(C),
        out_shape=jax.ShapeDtypeStruct((N, C), jnp.float32),
    )(acc3, rdeg, v3, bl3.reshape(1, C))

    return out


# speedup vs baseline: 4.8978x; 1.2128x over previous
import functools

import jax
import jax.numpy as jnp
from jax import lax
from jax.experimental import pallas as pl
from jax.experimental.pallas import tpu as pltpu
from jax.experimental.pallas import tpu_sc as plsc

N = 10000
E = 320000
F = 128
H = 128
H2 = 256
C = 40

NC = 2
NS = 16
LANES = 16
NW = NC * NS

K = 128
E_PAD = 327680
EPT = E_PAD // NW
NCHUNK = EPT // K
EBUF = 4
NPAD = 10016
RPT = NPAD // NS

D1 = 144
D2 = 128
D3 = 48

BN = 400
GRID = N // BN

_PREC = lax.Precision.HIGHEST


def _dot(a, b):
    return jnp.dot(a, b, precision=_PREC, preferred_element_type=jnp.float32)


def _make_sc_pass(D, NBUF):
    mesh = plsc.VectorSubcoreMesh(core_axis_name="c", subcore_axis_name="s")
    STEP = 4

    @functools.partial(
        pl.kernel,
        mesh=mesh,
        compiler_params=pltpu.CompilerParams(use_tc_tiling_on_sc=False),
        out_type=jax.ShapeDtypeStruct((NC, NPAD, D), jnp.float32),
        scratch_types=[
            pltpu.VMEM((EBUF, 2, K), jnp.int32),
            pltpu.VMEM((NBUF, K, D), jnp.float32),
            pltpu.VMEM_SHARED((NPAD, D), jnp.float32),
            pltpu.SemaphoreType.DMA((EBUF,)),
            pltpu.SemaphoreType.DMA((NBUF,)),
            pltpu.SemaphoreType.DMA((NBUF,)),
        ],
    )
    def sc_pass(table_hbm, edge_hbm, zero_hbm, out_hbm, ev, rows_v, acc_sh,
                esem, gsem, ssem):
        c = lax.axis_index("c")
        s = lax.axis_index("s")
        wid = c * NS + s
        ebase = wid * EPT
        zbase = s * RPT

        def edge_cp(ci, eb):
            return pltpu.make_async_copy(
                edge_hbm.at[:, pl.ds(ebase + ci * K, K)], ev.at[eb],
                esem.at[eb])

        def gather_cp(ci, eb, b):
            del ci
            return pltpu.make_async_copy(
                table_hbm.at[ev.at[eb].at[0]], rows_v.at[b], gsem.at[b])

        def scatter_dst(eb):
            return acc_sh.at[ev.at[eb].at[1]]

        pltpu.sync_copy(zero_hbm, acc_sh.at[pl.ds(zbase, RPT)])

        edge_cp(0, 0).start()
        edge_cp(1, 1).start()
        plsc.subcore_barrier()
        edge_cp(0, 0).wait()
        gather_cp(0, 0, 0).start()

        @pl.loop(0, NCHUNK, step=STEP)
        def _chunk(ci0):
            for j in range(STEP):
                ci = ci0 + j
                b = j % NBUF
                bp = (b + NBUF - 1) % NBUF
                bn = (j + 1) % NBUF
                eb = j % EBUF
                en = (j + 1) % EBUF
                ep = (j + 2) % EBUF
                ebp = (j + EBUF - 1) % EBUF

                @pl.when(ci >= 1)
                def _retire():
                    pltpu.make_async_copy(
                        rows_v.at[bp], scatter_dst(ebp), ssem.at[bp]).wait()

                @pl.when(ci + 2 < NCHUNK)
                def _eprefetch():
                    edge_cp(ci + 2, ep).start()

                @pl.when(ci + 1 < NCHUNK)
                def _gnext():
                    edge_cp(ci + 1, en).wait()
                    gather_cp(ci + 1, en, bn).start()

                gather_cp(ci, eb, b).wait()
                pltpu.async_copy(rows_v.at[b], scatter_dst(eb),
                                 ssem.at[b], add=True)

        lb = (NCHUNK - 1) % NBUF
        le = (NCHUNK - 1) % EBUF
        pltpu.make_async_copy(rows_v.at[lb], scatter_dst(le),
                              ssem.at[lb]).wait()

        plsc.subcore_barrier()

        pltpu.sync_copy(acc_sh.at[pl.ds(zbase, RPT)],
                        out_hbm.at[c].at[pl.ds(zbase, RPT)])

    return sc_pass


_sc_pass_d1 = _make_sc_pass(D1, 2)
_sc_pass_d2 = _make_sc_pass(D2, 2)
_sc_pass_d3 = _make_sc_pass(D3, 4)


def _row_spec(d):
    return pl.BlockSpec((BN, d), lambda i: (i, 0))


def _acc_spec(d):
    return pl.BlockSpec((NC, BN, d), lambda i: (0, i, 0))


def _full_spec(shape):
    return pl.BlockSpec(shape, lambda i: tuple(0 for _ in shape))


def _pre_body(x_ref, wmap_ref, bmap_ref, wl1_ref, wr1_ref,
              h0_ref, t1_ref, u1_ref):
    h0 = _dot(x_ref[...], wmap_ref[...]) + bmap_ref[...]
    h0_ref[...] = h0
    p1 = _dot(h0, wl1_ref[...])
    t1_ref[...] = jnp.concatenate(
        [p1, jnp.ones((BN, D1 - H), jnp.float32)], axis=1)
    u1_ref[...] = _dot(h0, wr1_ref[...])


def _l1_body(acc_ref, u1_ref, bl1_ref, h1_ref, rdeg_ref):
    ssum = acc_ref[0, :, :H] + acc_ref[1, :, :H]
    deg = acc_ref[0, :, H:H + 8] + acc_ref[1, :, H:H + 8]
    rdeg = 1.0 / jnp.maximum(deg[:, :1], 1.0)
    h1 = ssum * rdeg + bl1_ref[...] + u1_ref[...]
    h1_ref[...] = jnp.maximum(h1, 0.0)
    rdeg_ref[...] = jnp.broadcast_to(rdeg, (BN, 8))


def _mm_body(a_ref, w_ref, o_ref):
    o_ref[...] = _dot(a_ref[...], w_ref[...])


def _l2_body(acc_ref, rdeg_ref, v2_ref, wl2_ref, bl2_ref, wl3_ref,
             h2_ref, t3_ref):
    agg = (acc_ref[0] + acc_ref[1]) * rdeg_ref[:, :1]
    h2 = _dot(agg, wl2_ref[...]) + bl2_ref[...] + v2_ref[...]
    h2 = jnp.maximum(h2, 0.0)
    h2_ref[...] = h2
    p3 = _dot(h2, wl3_ref[...])
    t3_ref[...] = jnp.concatenate(
        [p3, jnp.zeros((BN, D3 - C), jnp.float32)], axis=1)


def _l3_body(acc_ref, rdeg_ref, v3_ref, bl3_ref, out_ref):
    aggs = acc_ref[0] + acc_ref[1]
    z = aggs[:, :C] * rdeg_ref[:, :1] + bl3_ref[...] + v3_ref[...]
    m = jnp.max(z, axis=1, keepdims=True)
    lse = jnp.log(jnp.sum(jnp.exp(z - m), axis=1, keepdims=True)) + m
    out_ref[...] = z - lse


def kernel(x, edge_index, W_map, b_map, Wl1, bl1, Wr1, Wl2, bl2, Wr2,
           Wl3, bl3, Wr3):
    edges = edge_index.astype(jnp.int32)
    pad = jnp.tile(jnp.array([[0], [NPAD - 1]], jnp.int32), (1, E_PAD - E))
    edges = jnp.concatenate([edges, pad], axis=1)
    z1 = jnp.zeros((RPT, D1), jnp.float32)
    z2 = jnp.zeros((RPT, D2), jnp.float32)
    z3 = jnp.zeros((RPT, D3), jnp.float32)

    h0, t1, u1 = pl.pallas_call(
        _pre_body,
        grid=(GRID,),
        in_specs=[_row_spec(F), _full_spec((F, H)), _full_spec((1, H)),
                  _full_spec((H, H)), _full_spec((H, H))],
        out_specs=[_row_spec(H), _row_spec(D1), _row_spec(H)],
        out_shape=[jax.ShapeDtypeStruct((N, H), jnp.float32),
                   jax.ShapeDtypeStruct((N, D1), jnp.float32),
                   jax.ShapeDtypeStruct((N, H), jnp.float32)],
    )(x, W_map, b_map.reshape(1, H), Wl1, Wr1)

    acc1 = _sc_pass_d1(t1, edges, z1)

    h1, rdeg = pl.pallas_call(
        _l1_body,
        grid=(GRID,),
        in_specs=[_acc_spec(D1), _row_spec(H), _full_spec((1, H))],
        out_specs=[_row_spec(H), _row_spec(8)],
        out_shape=[jax.ShapeDtypeStruct((N, H), jnp.float32),
                   jax.ShapeDtypeStruct((N, 8), jnp.float32)],
    )(acc1, u1, bl1.reshape(1, H))

    acc2 = _sc_pass_d2(h1, edges, z2)

    v2 = pl.pallas_call(
        _mm_body,
        grid=(GRID,),
        in_specs=[_row_spec(H), _full_spec((H, H2))],
        out_specs=_row_spec(H2),
        out_shape=jax.ShapeDtypeStruct((N, H2), jnp.float32),
    )(h1, Wr2)

    h2, t3 = pl.pallas_call(
        _l2_body,
        grid=(GRID,),
        in_specs=[_acc_spec(D2), _row_spec(8), _row_spec(H2),
                  _full_spec((H, H2)), _full_spec((1, H2)),
                  _full_spec((H2, C))],
        out_specs=[_row_spec(H2), _row_spec(D3)],
        out_shape=[jax.ShapeDtypeStruct((N, H2), jnp.float32),
                   jax.ShapeDtypeStruct((N, D3), jnp.float32)],
    )(acc2, rdeg, v2, Wl2, bl2.reshape(1, H2), Wl3)

    acc3 = _sc_pass_d3(t3, edges, z3)

    v3 = pl.pallas_call(
        _mm_body,
        grid=(GRID,),
        in_specs=[_row_spec(H2), _full_spec((H2, C))],
        out_specs=_row_spec(C),
        out_shape=jax.ShapeDtypeStruct((N, C), jnp.float32),
    )(h2, Wr3)

    out = pl.pallas_call(
        _l3_body,
        grid=(GRID,),
        in_specs=[_acc_spec(D3), _row_spec(8), _row_spec(C),
                  _full_spec((1, C))],
        out_specs=_row_spec(C),
        out_shape=jax.ShapeDtypeStruct((N, C), jnp.float32),
    )(acc3, rdeg, v3, bl3.reshape(1, C))

    return out


# speedup vs baseline: 11.8624x; 2.4220x over previous
import functools

import jax
import jax.numpy as jnp
from jax import lax
from jax.experimental import pallas as pl
from jax.experimental.pallas import tpu as pltpu
from jax.experimental.pallas import tpu_sc as plsc

N = 10000
E = 320000
F = 128
H = 128
H2 = 256
C = 40

NC = 2
NS = 16
LANES = 16
NW = NC * NS

K = 80
EPT = E // NW
NCHUNK = EPT // K
EBUF = 4
STEP = 12
NLOOP = 132
NPAD = 10000
RPT = NPAD // NS

D1 = 144
D2 = 128
D3 = 48

BN = 400
GRID = N // BN

_PREC = lax.Precision.HIGHEST


def _dot(a, b):
    return jnp.dot(a, b, precision=_PREC, preferred_element_type=jnp.float32)


def _make_sc_pass(D, NBUF):
    mesh = plsc.VectorSubcoreMesh(core_axis_name="c", subcore_axis_name="s")
    assert STEP % NBUF == 0 and STEP % EBUF == 0

    @functools.partial(
        pl.kernel,
        mesh=mesh,
        compiler_params=pltpu.CompilerParams(use_tc_tiling_on_sc=False),
        out_type=jax.ShapeDtypeStruct((NC, NPAD, D), jnp.float32),
        scratch_types=[
            pltpu.VMEM((EBUF, 2, K), jnp.int32),
            pltpu.VMEM((NBUF, K, D), jnp.float32),
            pltpu.VMEM_SHARED((NPAD, D), jnp.float32),
            pltpu.SemaphoreType.DMA((EBUF,)),
            pltpu.SemaphoreType.DMA((NBUF,)),
            pltpu.SemaphoreType.DMA((NBUF,)),
        ],
    )
    def sc_pass(table_hbm, edge_hbm, zero_hbm, out_hbm, ev, rows_v, acc_sh,
                esem, gsem, ssem):
        c = lax.axis_index("c")
        s = lax.axis_index("s")
        wid = c * NS + s
        ebase = wid * EPT
        zbase = s * RPT

        def edge_cp(ci, eb):
            return pltpu.make_async_copy(
                edge_hbm.at[:, pl.ds(ebase + ci * K, K)], ev.at[eb],
                esem.at[eb])

        def gather_cp(ci, eb, b):
            del ci
            return pltpu.make_async_copy(
                table_hbm.at[ev.at[eb].at[0]], rows_v.at[b], gsem.at[b])

        def scatter_dst(eb):
            return acc_sh.at[ev.at[eb].at[1]]

        pltpu.sync_copy(zero_hbm, acc_sh.at[pl.ds(zbase, RPT)])

        edge_cp(0, 0).start()
        edge_cp(1, 1).start()
        plsc.subcore_barrier()
        edge_cp(0, 0).wait()
        gather_cp(0, 0, 0).start()

        @pl.loop(0, NLOOP, step=STEP)
        def _chunk(ci0):
            for j in range(STEP):
                ci = ci0 + j
                b = j % NBUF
                bp = (b + NBUF - 1) % NBUF
                bn = (j + 1) % NBUF
                eb = j % EBUF
                en = (j + 1) % EBUF
                ep = (j + 2) % EBUF
                ebp = (j + EBUF - 1) % EBUF

                @pl.when(jnp.logical_and(ci >= 1, ci <= NCHUNK))
                def _retire():
                    pltpu.make_async_copy(
                        rows_v.at[bp], scatter_dst(ebp), ssem.at[bp]).wait()

                @pl.when(ci + 2 < NCHUNK)
                def _eprefetch():
                    edge_cp(ci + 2, ep).start()

                @pl.when(ci + 1 < NCHUNK)
                def _gnext():
                    edge_cp(ci + 1, en).wait()
                    gather_cp(ci + 1, en, bn).start()

                @pl.when(ci < NCHUNK)
                def _scatter():
                    gather_cp(ci, eb, b).wait()
                    pltpu.async_copy(rows_v.at[b], scatter_dst(eb),
                                     ssem.at[b], add=True)

        plsc.subcore_barrier()

        pltpu.sync_copy(acc_sh.at[pl.ds(zbase, RPT)],
                        out_hbm.at[c].at[pl.ds(zbase, RPT)])

    return sc_pass


_sc_pass_d1 = _make_sc_pass(D1, 3)
_sc_pass_d2 = _make_sc_pass(D2, 4)
_sc_pass_d3 = _make_sc_pass(D3, 4)


def _row_spec(d):
    return pl.BlockSpec((BN, d), lambda i: (i, 0))


def _acc_spec(d):
    return pl.BlockSpec((NC, BN, d), lambda i: (0, i, 0))


def _full_spec(shape):
    return pl.BlockSpec(shape, lambda i: tuple(0 for _ in shape))


def _pre_body(x_ref, wmap_ref, bmap_ref, wl1_ref, wr1_ref,
              h0_ref, t1_ref, u1_ref):
    h0 = _dot(x_ref[...], wmap_ref[...]) + bmap_ref[...]
    h0_ref[...] = h0
    p1 = _dot(h0, wl1_ref[...])
    t1_ref[...] = jnp.concatenate(
        [p1, jnp.ones((BN, D1 - H), jnp.float32)], axis=1)
    u1_ref[...] = _dot(h0, wr1_ref[...])


def _l1_body(acc_ref, u1_ref, bl1_ref, h1_ref, rdeg_ref):
    ssum = acc_ref[0, :, :H] + acc_ref[1, :, :H]
    deg = acc_ref[0, :, H:H + 8] + acc_ref[1, :, H:H + 8]
    rdeg = 1.0 / jnp.maximum(deg[:, :1], 1.0)
    h1 = ssum * rdeg + bl1_ref[...] + u1_ref[...]
    h1_ref[...] = jnp.maximum(h1, 0.0)
    rdeg_ref[...] = jnp.broadcast_to(rdeg, (BN, 8))


def _mm_body(a_ref, w_ref, o_ref):
    o_ref[...] = _dot(a_ref[...], w_ref[...])


def _l2_body(acc_ref, rdeg_ref, v2_ref, wl2_ref, bl2_ref, wl3_ref,
             h2_ref, t3_ref):
    agg = (acc_ref[0] + acc_ref[1]) * rdeg_ref[:, :1]
    h2 = _dot(agg, wl2_ref[...]) + bl2_ref[...] + v2_ref[...]
    h2 = jnp.maximum(h2, 0.0)
    h2_ref[...] = h2
    p3 = _dot(h2, wl3_ref[...])
    t3_ref[...] = jnp.concatenate(
        [p3, jnp.zeros((BN, D3 - C), jnp.float32)], axis=1)


def _l3_body(acc_ref, rdeg_ref, v3_ref, bl3_ref, out_ref):
    aggs = acc_ref[0] + acc_ref[1]
    z = aggs[:, :C] * rdeg_ref[:, :1] + bl3_ref[...] + v3_ref[...]
    m = jnp.max(z, axis=1, keepdims=True)
    lse = jnp.log(jnp.sum(jnp.exp(z - m), axis=1, keepdims=True)) + m
    out_ref[...] = z - lse


def kernel(x, edge_index, W_map, b_map, Wl1, bl1, Wr1, Wl2, bl2, Wr2,
           Wl3, bl3, Wr3):
    edges = edge_index.astype(jnp.int32)
    z1 = jnp.zeros((RPT, D1), jnp.float32)
    z2 = jnp.zeros((RPT, D2), jnp.float32)
    z3 = jnp.zeros((RPT, D3), jnp.float32)

    h0, t1, u1 = pl.pallas_call(
        _pre_body,
        grid=(GRID,),
        in_specs=[_row_spec(F), _full_spec((F, H)), _full_spec((1, H)),
                  _full_spec((H, H)), _full_spec((H, H))],
        out_specs=[_row_spec(H), _row_spec(D1), _row_spec(H)],
        out_shape=[jax.ShapeDtypeStruct((N, H), jnp.float32),
                   jax.ShapeDtypeStruct((N, D1), jnp.float32),
                   jax.ShapeDtypeStruct((N, H), jnp.float32)],
    )(x, W_map, b_map.reshape(1, H), Wl1, Wr1)

    acc1 = _sc_pass_d1(t1, edges, z1)

    h1, rdeg = pl.pallas_call(
        _l1_body,
        grid=(GRID,),
        in_specs=[_acc_spec(D1), _row_spec(H), _full_spec((1, H))],
        out_specs=[_row_spec(H), _row_spec(8)],
        out_shape=[jax.ShapeDtypeStruct((N, H), jnp.float32),
                   jax.ShapeDtypeStruct((N, 8), jnp.float32)],
    )(acc1, u1, bl1.reshape(1, H))

    acc2 = _sc_pass_d2(h1, edges, z2)

    v2 = pl.pallas_call(
        _mm_body,
        grid=(GRID,),
        in_specs=[_row_spec(H), _full_spec((H, H2))],
        out_specs=_row_spec(H2),
        out_shape=jax.ShapeDtypeStruct((N, H2), jnp.float32),
    )(h1, Wr2)

    h2, t3 = pl.pallas_call(
        _l2_body,
        grid=(GRID,),
        in_specs=[_acc_spec(D2), _row_spec(8), _row_spec(H2),
                  _full_spec((H, H2)), _full_spec((1, H2)),
                  _full_spec((H2, C))],
        out_specs=[_row_spec(H2), _row_spec(D3)],
        out_shape=[jax.ShapeDtypeStruct((N, H2), jnp.float32),
                   jax.ShapeDtypeStruct((N, D3), jnp.float32)],
    )(acc2, rdeg, v2, Wl2, bl2.reshape(1, H2), Wl3)

    acc3 = _sc_pass_d3(t3, edges, z3)

    v3 = pl.pallas_call(
        _mm_body,
        grid=(GRID,),
        in_specs=[_row_spec(H2), _full_spec((H2, C))],
        out_specs=_row_spec(C),
        out_shape=jax.ShapeDtypeStruct((N, C), jnp.float32),
    )(h2, Wr3)

    out = pl.pallas_call(
        _l3_body,
        grid=(GRID,),
        in_specs=[_acc_spec(D3), _row_spec(8), _row_spec(C),
                  _full_spec((1, C))],
        out_specs=_row_spec(C),
        out_shape=jax.ShapeDtypeStruct((N, C), jnp.float32),
    )(acc3, rdeg, v3, bl3.reshape(1, C))

    return out


# speedup vs baseline: 12.6949x; 1.0702x over previous
import functools

import jax
import jax.numpy as jnp
from jax import lax
from jax.experimental import pallas as pl
from jax.experimental.pallas import tpu as pltpu
from jax.experimental.pallas import tpu_sc as plsc

N = 10000
E = 320000
F = 128
H = 128
H2 = 256
C = 40

NC = 2
NS = 16
LANES = 16
NW = NC * NS

K = 80
EPT = E // NW
NCHUNK = EPT // K
EBUF = 4
STEP = 12
NLOOP = 132
NPAD = 10000
RPT = NPAD // NS

D1 = 144
D2 = 128
D3 = 48

BN = 2000
GRID = N // BN

_PREC = lax.Precision.HIGHEST


def _dot(a, b):
    return jnp.dot(a, b, precision=_PREC, preferred_element_type=jnp.float32)


def _make_sc_pass(D, NBUF):
    mesh = plsc.VectorSubcoreMesh(core_axis_name="c", subcore_axis_name="s")
    assert STEP % NBUF == 0 and STEP % EBUF == 0

    @functools.partial(
        pl.kernel,
        mesh=mesh,
        compiler_params=pltpu.CompilerParams(use_tc_tiling_on_sc=False),
        out_type=jax.ShapeDtypeStruct((NC, NPAD, D), jnp.float32),
        scratch_types=[
            pltpu.VMEM((EBUF, 2, K), jnp.int32),
            pltpu.VMEM((NBUF, K, D), jnp.float32),
            pltpu.VMEM_SHARED((NPAD, D), jnp.float32),
            pltpu.SemaphoreType.DMA((EBUF,)),
            pltpu.SemaphoreType.DMA((NBUF,)),
            pltpu.SemaphoreType.DMA((NBUF,)),
        ],
    )
    def sc_pass(table_hbm, edge_hbm, zero_hbm, out_hbm, ev, rows_v, acc_sh,
                esem, gsem, ssem):
        c = lax.axis_index("c")
        s = lax.axis_index("s")
        wid = c * NS + s
        ebase = wid * EPT
        zbase = s * RPT

        def edge_cp(ci, eb):
            return pltpu.make_async_copy(
                edge_hbm.at[:, pl.ds(ebase + ci * K, K)], ev.at[eb],
                esem.at[eb])

        def gather_cp(ci, eb, b):
            del ci
            return pltpu.make_async_copy(
                table_hbm.at[ev.at[eb].at[0]], rows_v.at[b], gsem.at[b])

        def scatter_dst(eb):
            return acc_sh.at[ev.at[eb].at[1]]

        pltpu.sync_copy(zero_hbm, acc_sh.at[pl.ds(zbase, RPT)])

        edge_cp(0, 0).start()
        edge_cp(1, 1).start()
        plsc.subcore_barrier()
        edge_cp(0, 0).wait()
        gather_cp(0, 0, 0).start()

        @pl.loop(0, NLOOP, step=STEP)
        def _chunk(ci0):
            for j in range(STEP):
                ci = ci0 + j
                b = j % NBUF
                bp = (b + NBUF - 1) % NBUF
                bn = (j + 1) % NBUF
                eb = j % EBUF
                en = (j + 1) % EBUF
                ep = (j + 2) % EBUF
                ebp = (j + EBUF - 1) % EBUF

                @pl.when(jnp.logical_and(ci >= 1, ci <= NCHUNK))
                def _retire():
                    pltpu.make_async_copy(
                        rows_v.at[bp], scatter_dst(ebp), ssem.at[bp]).wait()

                @pl.when(ci + 2 < NCHUNK)
                def _eprefetch():
                    edge_cp(ci + 2, ep).start()

                @pl.when(ci + 1 < NCHUNK)
                def _gnext():
                    edge_cp(ci + 1, en).wait()
                    gather_cp(ci + 1, en, bn).start()

                @pl.when(ci < NCHUNK)
                def _scatter():
                    gather_cp(ci, eb, b).wait()
                    pltpu.async_copy(rows_v.at[b], scatter_dst(eb),
                                     ssem.at[b], add=True)

        plsc.subcore_barrier()

        pltpu.sync_copy(acc_sh.at[pl.ds(zbase, RPT)],
                        out_hbm.at[c].at[pl.ds(zbase, RPT)])

    return sc_pass


_sc_pass_d1 = _make_sc_pass(D1, 3)
_sc_pass_d2 = _make_sc_pass(D2, 4)
_sc_pass_d3 = _make_sc_pass(D3, 4)


def _row_spec(d):
    return pl.BlockSpec((BN, d), lambda i: (i, 0))


def _acc_spec(d):
    return pl.BlockSpec((NC, BN, d), lambda i: (0, i, 0))


def _full_spec(shape):
    return pl.BlockSpec(shape, lambda i: tuple(0 for _ in shape))


def _pre_body(x_ref, wmap_ref, bmap_ref, wl1_ref, wr1_ref,
              h0_ref, t1_ref, u1_ref):
    h0 = _dot(x_ref[...], wmap_ref[...]) + bmap_ref[...]
    h0_ref[...] = h0
    p1 = _dot(h0, wl1_ref[...])
    t1_ref[...] = jnp.concatenate(
        [p1, jnp.ones((BN, D1 - H), jnp.float32)], axis=1)
    u1_ref[...] = _dot(h0, wr1_ref[...])


def _l1_body(acc_ref, u1_ref, bl1_ref, h1_ref, rdeg_ref):
    ssum = acc_ref[0, :, :H] + acc_ref[1, :, :H]
    deg = acc_ref[0, :, H:H + 8] + acc_ref[1, :, H:H + 8]
    rdeg = 1.0 / jnp.maximum(deg[:, :1], 1.0)
    h1 = ssum * rdeg + bl1_ref[...] + u1_ref[...]
    h1_ref[...] = jnp.maximum(h1, 0.0)
    rdeg_ref[...] = jnp.broadcast_to(rdeg, (BN, 8))


def _mm_body(a_ref, w_ref, o_ref):
    o_ref[...] = _dot(a_ref[...], w_ref[...])


def _l2_body(acc_ref, rdeg_ref, v2_ref, wl2_ref, bl2_ref, wl3_ref,
             h2_ref, t3_ref):
    agg = (acc_ref[0] + acc_ref[1]) * rdeg_ref[:, :1]
    h2 = _dot(agg, wl2_ref[...]) + bl2_ref[...] + v2_ref[...]
    h2 = jnp.maximum(h2, 0.0)
    h2_ref[...] = h2
    p3 = _dot(h2, wl3_ref[...])
    t3_ref[...] = jnp.concatenate(
        [p3, jnp.zeros((BN, D3 - C), jnp.float32)], axis=1)


def _l3_body(acc_ref, rdeg_ref, v3_ref, bl3_ref, out_ref):
    aggs = acc_ref[0] + acc_ref[1]
    z = aggs[:, :C] * rdeg_ref[:, :1] + bl3_ref[...] + v3_ref[...]
    m = jnp.max(z, axis=1, keepdims=True)
    lse = jnp.log(jnp.sum(jnp.exp(z - m), axis=1, keepdims=True)) + m
    out_ref[...] = z - lse


def kernel(x, edge_index, W_map, b_map, Wl1, bl1, Wr1, Wl2, bl2, Wr2,
           Wl3, bl3, Wr3):
    edges = edge_index.astype(jnp.int32)
    z1 = jnp.zeros((RPT, D1), jnp.float32)
    z2 = jnp.zeros((RPT, D2), jnp.float32)
    z3 = jnp.zeros((RPT, D3), jnp.float32)

    h0, t1, u1 = pl.pallas_call(
        _pre_body,
        grid=(GRID,),
        in_specs=[_row_spec(F), _full_spec((F, H)), _full_spec((1, H)),
                  _full_spec((H, H)), _full_spec((H, H))],
        out_specs=[_row_spec(H), _row_spec(D1), _row_spec(H)],
        out_shape=[jax.ShapeDtypeStruct((N, H), jnp.float32),
                   jax.ShapeDtypeStruct((N, D1), jnp.float32),
                   jax.ShapeDtypeStruct((N, H), jnp.float32)],
    )(x, W_map, b_map.reshape(1, H), Wl1, Wr1)

    acc1 = _sc_pass_d1(t1, edges, z1)

    h1, rdeg = pl.pallas_call(
        _l1_body,
        grid=(GRID,),
        in_specs=[_acc_spec(D1), _row_spec(H), _full_spec((1, H))],
        out_specs=[_row_spec(H), _row_spec(8)],
        out_shape=[jax.ShapeDtypeStruct((N, H), jnp.float32),
                   jax.ShapeDtypeStruct((N, 8), jnp.float32)],
    )(acc1, u1, bl1.reshape(1, H))

    acc2 = _sc_pass_d2(h1, edges, z2)

    v2 = pl.pallas_call(
        _mm_body,
        grid=(GRID,),
        in_specs=[_row_spec(H), _full_spec((H, H2))],
        out_specs=_row_spec(H2),
        out_shape=jax.ShapeDtypeStruct((N, H2), jnp.float32),
    )(h1, Wr2)

    h2, t3 = pl.pallas_call(
        _l2_body,
        grid=(GRID,),
        in_specs=[_acc_spec(D2), _row_spec(8), _row_spec(H2),
                  _full_spec((H, H2)), _full_spec((1, H2)),
                  _full_spec((H2, C))],
        out_specs=[_row_spec(H2), _row_spec(D3)],
        out_shape=[jax.ShapeDtypeStruct((N, H2), jnp.float32),
                   jax.ShapeDtypeStruct((N, D3), jnp.float32)],
    )(acc2, rdeg, v2, Wl2, bl2.reshape(1, H2), Wl3)

    acc3 = _sc_pass_d3(t3, edges, z3)

    v3 = pl.pallas_call(
        _mm_body,
        grid=(GRID,),
        in_specs=[_row_spec(H2), _full_spec((H2, C))],
        out_specs=_row_spec(C),
        out_shape=jax.ShapeDtypeStruct((N, C), jnp.float32),
    )(h2, Wr3)

    out = pl.pallas_call(
        _l3_body,
        grid=(GRID,),
        in_specs=[_acc_spec(D3), _row_spec(8), _row_spec(C),
                  _full_spec((1, C))],
        out_specs=_row_spec(C),
        out_shape=jax.ShapeDtypeStruct((N, C), jnp.float32),
    )(acc3, rdeg, v3, bl3.reshape(1, C))

    return out


# speedup vs baseline: 13.1354x; 1.0347x over previous
import functools

import jax
import jax.numpy as jnp
from jax import lax
from jax.experimental import pallas as pl
from jax.experimental.pallas import tpu as pltpu
from jax.experimental.pallas import tpu_sc as plsc

N = 10000
E = 320000
F = 128
H = 128
H2 = 256
C = 40

NC = 2
NS = 16
LANES = 16
NW = NC * NS

K = 80
EPT = E // NW
NCHUNK = EPT // K
EBUF = 4
STEP = 12
NLOOP = 132
NPAD = 10000
RPT = NPAD // NS

D1 = 144
D2 = 128
D3 = 48

BN = 2000
GRID = N // BN

def _dot(a, b):
    a16 = a.astype(jnp.bfloat16)
    b16 = b.astype(jnp.bfloat16)
    ar = (a - a16.astype(jnp.float32)).astype(jnp.bfloat16)
    br = (b - b16.astype(jnp.float32)).astype(jnp.bfloat16)
    d = jnp.dot(a16, b16, preferred_element_type=jnp.float32)
    d = d + jnp.dot(ar, b16, preferred_element_type=jnp.float32)
    d = d + jnp.dot(a16, br, preferred_element_type=jnp.float32)
    return d


def _make_sc_pass(D, NBUF):
    mesh = plsc.VectorSubcoreMesh(core_axis_name="c", subcore_axis_name="s")
    assert STEP % NBUF == 0 and STEP % EBUF == 0

    @functools.partial(
        pl.kernel,
        mesh=mesh,
        compiler_params=pltpu.CompilerParams(use_tc_tiling_on_sc=False),
        out_type=jax.ShapeDtypeStruct((NC, NPAD, D), jnp.float32),
        scratch_types=[
            pltpu.VMEM((EBUF, 2, K), jnp.int32),
            pltpu.VMEM((NBUF, K, D), jnp.float32),
            pltpu.VMEM_SHARED((NPAD, D), jnp.float32),
            pltpu.SemaphoreType.DMA((EBUF,)),
            pltpu.SemaphoreType.DMA((NBUF,)),
            pltpu.SemaphoreType.DMA((NBUF,)),
        ],
    )
    def sc_pass(table_hbm, edge_hbm, zero_hbm, out_hbm, ev, rows_v, acc_sh,
                esem, gsem, ssem):
        c = lax.axis_index("c")
        s = lax.axis_index("s")
        wid = c * NS + s
        ebase = wid * EPT
        zbase = s * RPT

        def edge_cp(ci, eb):
            return pltpu.make_async_copy(
                edge_hbm.at[:, pl.ds(ebase + ci * K, K)], ev.at[eb],
                esem.at[eb])

        def gather_cp(ci, eb, b):
            del ci
            return pltpu.make_async_copy(
                table_hbm.at[ev.at[eb].at[0]], rows_v.at[b], gsem.at[b])

        def scatter_dst(eb):
            return acc_sh.at[ev.at[eb].at[1]]

        pltpu.sync_copy(zero_hbm, acc_sh.at[pl.ds(zbase, RPT)])

        edge_cp(0, 0).start()
        edge_cp(1, 1).start()
        plsc.subcore_barrier()
        edge_cp(0, 0).wait()
        gather_cp(0, 0, 0).start()

        @pl.loop(0, NLOOP, step=STEP)
        def _chunk(ci0):
            for j in range(STEP):
                ci = ci0 + j
                b = j % NBUF
                bp = (b + NBUF - 1) % NBUF
                bn = (j + 1) % NBUF
                eb = j % EBUF
                en = (j + 1) % EBUF
                ep = (j + 2) % EBUF
                ebp = (j + EBUF - 1) % EBUF

                @pl.when(jnp.logical_and(ci >= 1, ci <= NCHUNK))
                def _retire():
                    pltpu.make_async_copy(
                        rows_v.at[bp], scatter_dst(ebp), ssem.at[bp]).wait()

                @pl.when(ci + 2 < NCHUNK)
                def _eprefetch():
                    edge_cp(ci + 2, ep).start()

                @pl.when(ci + 1 < NCHUNK)
                def _gnext():
                    edge_cp(ci + 1, en).wait()
                    gather_cp(ci + 1, en, bn).start()

                @pl.when(ci < NCHUNK)
                def _scatter():
                    gather_cp(ci, eb, b).wait()
                    pltpu.async_copy(rows_v.at[b], scatter_dst(eb),
                                     ssem.at[b], add=True)

        plsc.subcore_barrier()

        pltpu.sync_copy(acc_sh.at[pl.ds(zbase, RPT)],
                        out_hbm.at[c].at[pl.ds(zbase, RPT)])

    return sc_pass


_sc_pass_d1 = _make_sc_pass(D1, 3)
_sc_pass_d2 = _make_sc_pass(D2, 4)
_sc_pass_d3 = _make_sc_pass(D3, 6)


def _row_spec(d):
    return pl.BlockSpec((BN, d), lambda i: (i, 0))


def _acc_spec(d):
    return pl.BlockSpec((NC, BN, d), lambda i: (0, i, 0))


def _full_spec(shape):
    return pl.BlockSpec(shape, lambda i: tuple(0 for _ in shape))


def _pre_body(x_ref, wmap_ref, bmap_ref, wl1_ref, wr1_ref,
              h0_ref, t1_ref, u1_ref):
    h0 = _dot(x_ref[...], wmap_ref[...]) + bmap_ref[...]
    h0_ref[...] = h0
    p1 = _dot(h0, wl1_ref[...])
    t1_ref[...] = jnp.concatenate(
        [p1, jnp.ones((BN, D1 - H), jnp.float32)], axis=1)
    u1_ref[...] = _dot(h0, wr1_ref[...])


def _l1_body(acc_ref, u1_ref, bl1_ref, h1_ref, rdeg_ref):
    ssum = acc_ref[0, :, :H] + acc_ref[1, :, :H]
    deg = acc_ref[0, :, H:H + 8] + acc_ref[1, :, H:H + 8]
    rdeg = 1.0 / jnp.maximum(deg[:, :1], 1.0)
    h1 = ssum * rdeg + bl1_ref[...] + u1_ref[...]
    h1_ref[...] = jnp.maximum(h1, 0.0)
    rdeg_ref[...] = jnp.broadcast_to(rdeg, (BN, 8))


def _mm_body(a_ref, w_ref, o_ref):
    o_ref[...] = _dot(a_ref[...], w_ref[...])


def _l2_body(acc_ref, rdeg_ref, v2_ref, wl2_ref, bl2_ref, wl3_ref,
             h2_ref, t3_ref):
    agg = (acc_ref[0] + acc_ref[1]) * rdeg_ref[:, :1]
    h2 = _dot(agg, wl2_ref[...]) + bl2_ref[...] + v2_ref[...]
    h2 = jnp.maximum(h2, 0.0)
    h2_ref[...] = h2
    p3 = _dot(h2, wl3_ref[...])
    t3_ref[...] = jnp.concatenate(
        [p3, jnp.zeros((BN, D3 - C), jnp.float32)], axis=1)


def _l3_body(acc_ref, rdeg_ref, v3_ref, bl3_ref, out_ref):
    aggs = acc_ref[0] + acc_ref[1]
    z = aggs[:, :C] * rdeg_ref[:, :1] + bl3_ref[...] + v3_ref[...]
    m = jnp.max(z, axis=1, keepdims=True)
    lse = jnp.log(jnp.sum(jnp.exp(z - m), axis=1, keepdims=True)) + m
    out_ref[...] = z - lse


def kernel(x, edge_index, W_map, b_map, Wl1, bl1, Wr1, Wl2, bl2, Wr2,
           Wl3, bl3, Wr3):
    edges = edge_index.astype(jnp.int32)
    z1 = jnp.zeros((RPT, D1), jnp.float32)
    z2 = jnp.zeros((RPT, D2), jnp.float32)
    z3 = jnp.zeros((RPT, D3), jnp.float32)

    h0, t1, u1 = pl.pallas_call(
        _pre_body,
        grid=(GRID,),
        in_specs=[_row_spec(F), _full_spec((F, H)), _full_spec((1, H)),
                  _full_spec((H, H)), _full_spec((H, H))],
        out_specs=[_row_spec(H), _row_spec(D1), _row_spec(H)],
        out_shape=[jax.ShapeDtypeStruct((N, H), jnp.float32),
                   jax.ShapeDtypeStruct((N, D1), jnp.float32),
                   jax.ShapeDtypeStruct((N, H), jnp.float32)],
    )(x, W_map, b_map.reshape(1, H), Wl1, Wr1)

    acc1 = _sc_pass_d1(t1, edges, z1)

    h1, rdeg = pl.pallas_call(
        _l1_body,
        grid=(GRID,),
        in_specs=[_acc_spec(D1), _row_spec(H), _full_spec((1, H))],
        out_specs=[_row_spec(H), _row_spec(8)],
        out_shape=[jax.ShapeDtypeStruct((N, H), jnp.float32),
                   jax.ShapeDtypeStruct((N, 8), jnp.float32)],
    )(acc1, u1, bl1.reshape(1, H))

    acc2 = _sc_pass_d2(h1, edges, z2)

    v2 = pl.pallas_call(
        _mm_body,
        grid=(GRID,),
        in_specs=[_row_spec(H), _full_spec((H, H2))],
        out_specs=_row_spec(H2),
        out_shape=jax.ShapeDtypeStruct((N, H2), jnp.float32),
    )(h1, Wr2)

    h2, t3 = pl.pallas_call(
        _l2_body,
        grid=(GRID,),
        in_specs=[_acc_spec(D2), _row_spec(8), _row_spec(H2),
                  _full_spec((H, H2)), _full_spec((1, H2)),
                  _full_spec((H2, C))],
        out_specs=[_row_spec(H2), _row_spec(D3)],
        out_shape=[jax.ShapeDtypeStruct((N, H2), jnp.float32),
                   jax.ShapeDtypeStruct((N, D3), jnp.float32)],
    )(acc2, rdeg, v2, Wl2, bl2.reshape(1, H2), Wl3)

    acc3 = _sc_pass_d3(t3, edges, z3)

    v3 = pl.pallas_call(
        _mm_body,
        grid=(GRID,),
        in_specs=[_row_spec(H2), _full_spec((H2, C))],
        out_specs=_row_spec(C),
        out_shape=jax.ShapeDtypeStruct((N, C), jnp.float32),
    )(h2, Wr3)

    out = pl.pallas_call(
        _l3_body,
        grid=(GRID,),
        in_specs=[_acc_spec(D3), _row_spec(8), _row_spec(C),
                  _full_spec((1, C))],
        out_specs=_row_spec(C),
        out_shape=jax.ShapeDtypeStruct((N, C), jnp.float32),
    )(acc3, rdeg, v3, bl3.reshape(1, C))

    return out


# speedup vs baseline: 14.2141x; 1.0821x over previous
import functools

import jax
import jax.numpy as jnp
from jax import lax
from jax.experimental import pallas as pl
from jax.experimental.pallas import tpu as pltpu
from jax.experimental.pallas import tpu_sc as plsc

N = 10000
E = 320000
F = 128
H = 128
H2 = 256
C = 40

NC = 2
NS = 16
LANES = 16
NW = NC * NS

K = 80
EPT = E // NW
NCHUNK = EPT // K
EBUF = 4
STEP = 12
NLOOP = 132
NPAD = 10000
RPT = NPAD // NS

D1 = 144
D2 = 128
D3 = 48

BN = 2000
GRID = N // BN

def _dot(a, b):
    a16 = a.astype(jnp.bfloat16)
    b16 = b.astype(jnp.bfloat16)
    ar = (a - a16.astype(jnp.float32)).astype(jnp.bfloat16)
    br = (b - b16.astype(jnp.float32)).astype(jnp.bfloat16)
    d = jnp.dot(a16, b16, preferred_element_type=jnp.float32)
    d = d + jnp.dot(ar, b16, preferred_element_type=jnp.float32)
    d = d + jnp.dot(a16, br, preferred_element_type=jnp.float32)
    return d


def _make_sc_pass(D, NBUF, with_deg=False):
    mesh = plsc.VectorSubcoreMesh(core_axis_name="c", subcore_axis_name="s")
    assert STEP % NBUF == 0 and STEP % EBUF == 0

    out_types = [jax.ShapeDtypeStruct((NC, NPAD, 128), jnp.float32)]
    scratch = [
        pltpu.VMEM((EBUF, 2, K), jnp.int32),
        pltpu.VMEM((NBUF, K, D), jnp.float32),
        pltpu.VMEM_SHARED((NPAD, D), jnp.float32),
        pltpu.SemaphoreType.DMA((EBUF,)),
        pltpu.SemaphoreType.DMA((NBUF,)),
        pltpu.SemaphoreType.DMA((NBUF,)),
    ]
    if with_deg:
        out_types.append(jax.ShapeDtypeStruct((NC, NPAD, 16), jnp.float32))
        scratch += [
            pltpu.VMEM((K, 16), jnp.float32),
            pltpu.VMEM_SHARED((NPAD, 16), jnp.float32),
            pltpu.SemaphoreType.DMA((NBUF,)),
        ]

    @functools.partial(
        pl.kernel,
        mesh=mesh,
        compiler_params=pltpu.CompilerParams(use_tc_tiling_on_sc=False),
        out_type=out_types if with_deg else out_types[0],
        scratch_types=scratch,
    )
    def sc_pass(table_hbm, edge_hbm, zero_hbm, out_hbm, *rest):
        if with_deg:
            (deg_hbm, ev, rows_v, acc_sh, esem, gsem, ssem,
             ones_v, dacc_sh, dsem) = rest
        else:
            ev, rows_v, acc_sh, esem, gsem, ssem = rest
        c = lax.axis_index("c")
        s = lax.axis_index("s")
        wid = c * NS + s
        ebase = wid * EPT
        zbase = s * RPT

        def edge_cp(ci, eb, half):
            return pltpu.make_async_copy(
                edge_hbm.at[pl.ds(half * E + ebase + ci * K, K)],
                ev.at[eb].at[half], esem.at[eb])

        def edge_start(ci, eb):
            edge_cp(ci, eb, 0).start()
            edge_cp(ci, eb, 1).start()

        def edge_wait(ci, eb):
            edge_cp(ci, eb, 0).wait()
            edge_cp(ci, eb, 1).wait()

        def gather_cp(eb, b):
            return pltpu.make_async_copy(
                table_hbm.at[ev.at[eb].at[0]], rows_v.at[b], gsem.at[b])

        def scatter_dst(eb):
            return acc_sh.at[ev.at[eb].at[1]]

        pltpu.sync_copy(zero_hbm.at[:, pl.ds(0, D)],
                        acc_sh.at[pl.ds(zbase, RPT)])
        if with_deg:
            pltpu.sync_copy(zero_hbm.at[:, pl.ds(0, 16)],
                            dacc_sh.at[pl.ds(zbase, RPT)])

            @pl.loop(0, K)
            def _ones(r):
                ones_v[r, :] = jnp.ones((16,), jnp.float32)

        edge_start(0, 0)
        edge_start(1, 1)
        plsc.subcore_barrier()
        edge_wait(0, 0)
        gather_cp(0, 0).start()

        @pl.loop(0, NLOOP, step=STEP)
        def _chunk(ci0):
            for j in range(STEP):
                ci = ci0 + j
                b = j % NBUF
                bp = (b + NBUF - 1) % NBUF
                bn = (j + 1) % NBUF
                eb = j % EBUF
                en = (j + 1) % EBUF
                ep = (j + 2) % EBUF
                ebp = (j + EBUF - 1) % EBUF

                @pl.when(jnp.logical_and(ci >= 1, ci <= NCHUNK))
                def _retire():
                    pltpu.make_async_copy(
                        rows_v.at[bp], scatter_dst(ebp), ssem.at[bp]).wait()
                    if with_deg:
                        pltpu.make_async_copy(
                            ones_v, dacc_sh.at[ev.at[ebp].at[1]],
                            dsem.at[bp]).wait()

                @pl.when(ci + 2 < NCHUNK)
                def _eprefetch():
                    edge_start(ci + 2, ep)

                @pl.when(ci + 1 < NCHUNK)
                def _gnext():
                    edge_wait(ci + 1, en)
                    gather_cp(en, bn).start()

                @pl.when(ci < NCHUNK)
                def _scatter():
                    gather_cp(eb, b).wait()
                    pltpu.async_copy(rows_v.at[b], scatter_dst(eb),
                                     ssem.at[b], add=True)
                    if with_deg:
                        pltpu.async_copy(ones_v,
                                         dacc_sh.at[ev.at[eb].at[1]],
                                         dsem.at[b], add=True)

        plsc.subcore_barrier()

        pltpu.sync_copy(acc_sh.at[pl.ds(zbase, RPT)],
                        out_hbm.at[c].at[pl.ds(zbase, RPT), pl.ds(0, D)])
        if with_deg:
            pltpu.sync_copy(dacc_sh.at[pl.ds(zbase, RPT)],
                            deg_hbm.at[c].at[pl.ds(zbase, RPT)])

    return sc_pass


_sc_pass_d1 = _make_sc_pass(128, 3, with_deg=True)
_sc_pass_d2 = _make_sc_pass(128, 4)
_sc_pass_d3 = _make_sc_pass(D3, 6)


def _row_spec(d):
    return pl.BlockSpec((BN, d), lambda i: (i, 0))


def _acc_spec(d):
    return pl.BlockSpec((NC, BN, d), lambda i: (0, i, 0))


def _full_spec(shape):
    return pl.BlockSpec(shape, lambda i: tuple(0 for _ in shape))


def _pre_body(x_ref, wmap_ref, bmap_ref, wl1_ref, wr1_ref,
              h0_ref, p1_ref, u1_ref):
    h0 = _dot(x_ref[...], wmap_ref[...]) + bmap_ref[...]
    h0_ref[...] = h0
    p1_ref[...] = _dot(h0, wl1_ref[...])
    u1_ref[...] = _dot(h0, wr1_ref[...])


def _l1_body(acc_ref, deg_ref, u1_ref, bl1_ref, h1_ref, rdeg_ref):
    ssum = acc_ref[0] + acc_ref[1]
    deg = deg_ref[0, :, :8] + deg_ref[1, :, :8]
    rdeg = 1.0 / jnp.maximum(deg[:, :1], 1.0)
    h1 = ssum * rdeg + bl1_ref[...] + u1_ref[...]
    h1_ref[...] = jnp.maximum(h1, 0.0)
    rdeg_ref[...] = jnp.broadcast_to(rdeg, (BN, 8))


def _mm_body(a_ref, w_ref, o_ref):
    o_ref[...] = _dot(a_ref[...], w_ref[...])


def _l2_body(acc_ref, rdeg_ref, v2_ref, wl2_ref, bl2_ref, wl3_ref,
             h2_ref, t3_ref):
    agg = (acc_ref[0] + acc_ref[1]) * rdeg_ref[:, :1]
    h2 = _dot(agg, wl2_ref[...]) + bl2_ref[...] + v2_ref[...]
    h2 = jnp.maximum(h2, 0.0)
    h2_ref[...] = h2
    p3 = _dot(h2, wl3_ref[...])
    t3_ref[...] = jnp.concatenate(
        [p3, jnp.zeros((BN, D3 - C), jnp.float32)], axis=1)


def _l3_body(acc_ref, rdeg_ref, v3_ref, bl3_ref, out_ref):
    aggs = acc_ref[0] + acc_ref[1]
    z = aggs[:, :C] * rdeg_ref[:, :1] + bl3_ref[...] + v3_ref[...]
    m = jnp.max(z, axis=1, keepdims=True)
    lse = jnp.log(jnp.sum(jnp.exp(z - m), axis=1, keepdims=True)) + m
    out_ref[...] = z - lse


def kernel(x, edge_index, W_map, b_map, Wl1, bl1, Wr1, Wl2, bl2, Wr2,
           Wl3, bl3, Wr3):
    edges = edge_index.astype(jnp.int32).reshape(2 * E)
    z = jnp.zeros((RPT, 128), jnp.float32)

    h0, p1, u1 = pl.pallas_call(
        _pre_body,
        grid=(GRID,),
        in_specs=[_row_spec(F), _full_spec((F, H)), _full_spec((1, H)),
                  _full_spec((H, H)), _full_spec((H, H))],
        out_specs=[_row_spec(H), _row_spec(H), _row_spec(H)],
        out_shape=[jax.ShapeDtypeStruct((N, H), jnp.float32),
                   jax.ShapeDtypeStruct((N, H), jnp.float32),
                   jax.ShapeDtypeStruct((N, H), jnp.float32)],
    )(x, W_map, b_map.reshape(1, H), Wl1, Wr1)

    acc1, deg16 = _sc_pass_d1(p1, edges, z)

    h1, rdeg = pl.pallas_call(
        _l1_body,
        grid=(GRID,),
        in_specs=[_acc_spec(128), _acc_spec(16), _row_spec(H),
                  _full_spec((1, H))],
        out_specs=[_row_spec(H), _row_spec(8)],
        out_shape=[jax.ShapeDtypeStruct((N, H), jnp.float32),
                   jax.ShapeDtypeStruct((N, 8), jnp.float32)],
    )(acc1, deg16, u1, bl1.reshape(1, H))

    acc2 = _sc_pass_d2(h1, edges, z)

    v2 = pl.pallas_call(
        _mm_body,
        grid=(GRID,),
        in_specs=[_row_spec(H), _full_spec((H, H2))],
        out_specs=_row_spec(H2),
        out_shape=jax.ShapeDtypeStruct((N, H2), jnp.float32),
    )(h1, Wr2)

    h2, t3 = pl.pallas_call(
        _l2_body,
        grid=(GRID,),
        in_specs=[_acc_spec(D2), _row_spec(8), _row_spec(H2),
                  _full_spec((H, H2)), _full_spec((1, H2)),
                  _full_spec((H2, C))],
        out_specs=[_row_spec(H2), _row_spec(D3)],
        out_shape=[jax.ShapeDtypeStruct((N, H2), jnp.float32),
                   jax.ShapeDtypeStruct((N, D3), jnp.float32)],
    )(acc2, rdeg, v2, Wl2, bl2.reshape(1, H2), Wl3)

    acc3 = _sc_pass_d3(t3, edges, z)

    v3 = pl.pallas_call(
        _mm_body,
        grid=(GRID,),
        in_specs=[_row_spec(H2), _full_spec((H2, C))],
        out_specs=_row_spec(C),
        out_shape=jax.ShapeDtypeStruct((N, C), jnp.float32),
    )(h2, Wr3)

    out = pl.pallas_call(
        _l3_body,
        grid=(GRID,),
        in_specs=[_acc_spec(128), _row_spec(8), _row_spec(C),
                  _full_spec((1, C))],
        out_specs=_row_spec(C),
        out_shape=jax.ShapeDtypeStruct((N, C), jnp.float32),
    )(acc3, rdeg, v3, bl3.reshape(1, C))

    return out
